# baseline (speedup 1.0000x reference)
import functools

import jax
import jax.numpy as jnp
from jax import lax
from jax.experimental import pallas as pl
from jax.experimental.pallas import tpu as pltpu
from jax.experimental.pallas import tpu_sc as plsc

N_NODES = 10000
N_EDGES = 320000
D_FEAT = 128
D_HID = 128
D_OUT = 64

NC = 2
NS = 16
NW = NC * NS

EPW = N_EDGES // NW
K = 80
NITER = EPW // K
N_PAD = 10240
RPT = N_PAD // NS
RB = 80
DCH = 8
SCH = 16
DEPTH = 4


@functools.lru_cache(maxsize=None)
def _make_sc_scatter(D):
    mesh = plsc.VectorSubcoreMesh(core_axis_name="c", subcore_axis_name="s")

    @functools.partial(
        pl.kernel,
        mesh=mesh,
        out_type=jax.ShapeDtypeStruct((NC, N_PAD, D), jnp.float32),
        scratch_types=[
            pltpu.VMEM((2, SCH, K), jnp.int32),
            pltpu.VMEM((2, DCH, K), jnp.int32),
            pltpu.VMEM((RB, D), jnp.float32),
            pltpu.VMEM((RB, D), jnp.float32),
            pltpu.VMEM((RB, D), jnp.float32),
            pltpu.VMEM((RB, D), jnp.float32),
            pltpu.VMEM_SHARED((N_PAD, D), jnp.float32),
            pltpu.SemaphoreType.DMA,
            pltpu.SemaphoreType.DMA,
            pltpu.SemaphoreType.DMA,
            pltpu.SemaphoreType.DMA,
            pltpu.SemaphoreType.DMA,
            pltpu.SemaphoreType.DMA,
        ],
    )
    def sc_kernel(y_hbm, src_hbm, dst_hbm, out_hbm,
                  src_v, dst_v, rows_a, rows_b, rows_c, rows_d, agg_sh,
                  sem_a, sem_b, sem_c, sem_dd, sem_i, sem_d):
        c = lax.axis_index("c")
        s = lax.axis_index("s")
        wid = c * NS + s
        zz = jnp.zeros((16,), jnp.float32)
        bufs = (rows_a, rows_b, rows_c, rows_d)
        sems = (sem_a, sem_b, sem_c, sem_dd)

        pltpu.sync_copy(src_hbm.at[wid, pl.ds(0, SCH)], src_v.at[0])
        pltpu.async_copy(src_hbm.at[wid, pl.ds(SCH, SCH)], src_v.at[1], sem_i)
        pltpu.async_copy(dst_hbm.at[wid, pl.ds(0, DCH)], dst_v.at[0], sem_d)

        @pl.loop(0, RB)
        def _(r):
            @pl.loop(0, D // 16)
            def _(j):
                rows_d[r, pl.ds(j * 16, 16)] = zz

        for b in range(RPT // RB):
            pltpu.async_copy(
                rows_d, agg_sh.at[pl.ds(s * RPT + b * RB, RB)], sem_dd)

        for t in range(DEPTH - 1):
            pltpu.async_copy(y_hbm.at[src_v.at[0, t]], bufs[t], sems[t])

        for b in range(RPT // RB):
            pltpu.make_async_copy(
                rows_d, agg_sh.at[pl.ds(s * RPT + b * RB, RB)], sem_dd).wait()
        pltpu.async_copy(y_hbm.at[src_v.at[0, DEPTH - 1]],
                         rows_d, sem_dd)
        plsc.subcore_barrier()

        NFULL = (NITER - 1) // DEPTH

        @pl.loop(0, NFULL)
        def _(j):
            for t in range(DEPTH):
                i = DEPTH * j + t
                buf, sem = bufs[t], sems[t]
                par = lax.rem(lax.div(i, DCH), 2)

                @pl.when(lax.rem(i, DCH) == 0)
                def _():
                    i8 = pl.multiple_of(i, DCH)
                    pltpu.make_async_copy(
                        dst_hbm.at[wid, pl.ds(i8, DCH)],
                        dst_v.at[par], sem_d).wait()

                    @pl.when(i + DCH < NITER)
                    def _():
                        pltpu.async_copy(
                            dst_hbm.at[wid, pl.ds(i8 + DCH, DCH)],
                            dst_v.at[1 - par], sem_d)

                g = i + DEPTH
                gpar = lax.rem(lax.div(g, SCH), 2)

                @pl.when((lax.rem(g, SCH) == 0) & (g < NITER))
                def _():
                    pltpu.make_async_copy(
                        src_hbm.at[wid, pl.ds(pl.multiple_of(g, SCH), SCH)],
                        src_v.at[gpar], sem_i).wait()

                @pl.when((lax.rem(g, SCH) == DEPTH) &
                         (g - DEPTH + SCH < NITER))
                def _():
                    g16 = pl.multiple_of(g - DEPTH, SCH)
                    pltpu.async_copy(
                        src_hbm.at[wid, pl.ds(g16 + SCH, SCH)],
                        src_v.at[1 - gpar], sem_i)

                ipar = lax.rem(lax.div(i, SCH), 2)
                pltpu.make_async_copy(
                    y_hbm.at[src_v.at[ipar, lax.rem(i, SCH)]], buf, sem).wait()
                pltpu.sync_copy(
                    buf, agg_sh.at[dst_v.at[par, lax.rem(i, DCH)]], add=True)

                @pl.when(g < NITER)
                def _():
                    pltpu.async_copy(
                        y_hbm.at[src_v.at[gpar, lax.rem(g, SCH)]], buf, sem)

        for t, i in ((0, NITER - 1),):
            buf, sem = bufs[t], sems[t]
            par = (i // DCH) % 2
            ipar = (i // SCH) % 2
            pltpu.make_async_copy(
                y_hbm.at[src_v.at[ipar, lax.rem(i, SCH)]], buf, sem).wait()
            pltpu.sync_copy(
                buf, agg_sh.at[dst_v.at[par, lax.rem(i, DCH)]], add=True)

        plsc.subcore_barrier()

        nwo = RPT // RB
        for b in range(nwo):
            r0 = s * RPT + b * RB
            pltpu.async_copy(
                agg_sh.at[pl.ds(r0, RB)], out_hbm.at[c, pl.ds(r0, RB)], sem_a)
        for b in range(nwo):
            r0 = s * RPT + b * RB
            pltpu.make_async_copy(
                agg_sh.at[pl.ds(r0, RB)], out_hbm.at[c, pl.ds(r0, RB)],
                sem_a).wait()

    return sc_kernel


_ROW_BLK = 5000
_GRID = N_NODES // _ROW_BLK


def _mid_body(x_ref, p_ref, b_ref, w_ref, o_ref):
    g = x_ref[...] + p_ref[0] + p_ref[1]
    z = jnp.dot(g, w_ref[...], preferred_element_type=jnp.float32) + b_ref[...]
    o_ref[...] = jnp.maximum(z, 0.0)


def _final_body(h_ref, q_ref, b_ref, w_ref, o_ref):
    g = h_ref[...] + q_ref[0] + q_ref[1]
    z = jnp.dot(g, w_ref[...], preferred_element_type=jnp.float32) + b_ref[...]
    m = jnp.max(z, axis=1, keepdims=True)
    lse = jnp.log(jnp.sum(jnp.exp(z - m), axis=1, keepdims=True)) + m
    o_ref[...] = z - lse


def _combine(body, x, p, b, w):
    n, d = x.shape
    dout = w.shape[1]
    return pl.pallas_call(
        body,
        grid=(_GRID,),
        in_specs=[
            pl.BlockSpec((_ROW_BLK, d), lambda i: (i, 0)),
            pl.BlockSpec((NC, _ROW_BLK, d), lambda i: (0, i, 0)),
            pl.BlockSpec((1, dout), lambda i: (0, 0)),
            pl.BlockSpec((d, dout), lambda i: (0, 0)),
        ],
        out_specs=pl.BlockSpec((_ROW_BLK, dout), lambda i: (i, 0)),
        out_shape=jax.ShapeDtypeStruct((n, dout), jnp.float32),
    )(x, p, b, w)


def kernel(features, edges, W1, b1, W2, b2):
    src = edges[0].astype(jnp.int32).reshape(NW, NITER, K)
    dst = edges[1].astype(jnp.int32).reshape(NW, NITER, K)
    b1r = b1.reshape(1, D_HID)
    b2r = b2.reshape(1, D_OUT)

    p = _make_sc_scatter(D_FEAT)(features, src, dst)
    h = _combine(_mid_body, features, p, b1r, W1)
    q = _make_sc_scatter(D_HID)(h, src, dst)
    return _combine(_final_body, h, q, b2r, W2)
